# baseline (speedup 1.0000x reference)
import functools

import jax
import jax.numpy as jnp
from jax import lax
from jax.experimental import pallas as pl
from jax.experimental.pallas import tpu as pltpu
from jax.experimental.pallas import tpu_sc as plsc

NC = 2
NS = 16
L = 16
NW = NC * NS
K = 32
CH = 128
TBLK = 2048


def _transpose_body(t_ref, o_ref):
    o_ref[...] = t_ref[...].T


def _to_row_major(table_t):
    n = table_t.shape[1]
    return pl.pallas_call(
        _transpose_body,
        grid=(pl.cdiv(n, TBLK),),
        in_specs=[pl.BlockSpec((K, TBLK), lambda j: (0, j))],
        out_specs=pl.BlockSpec((TBLK, K), lambda j: (j, 0)),
        out_shape=jax.ShapeDtypeStruct((n, K), jnp.float32),
    )(table_t)


def kernel(data_u, data_i, u_table, i_table):
    B = data_u.shape[0]
    bw = B // NW
    mesh = plsc.VectorSubcoreMesh(core_axis_name="c", subcore_axis_name="s")

    @pl.kernel(
        mesh=mesh,
        out_type=jax.ShapeDtypeStruct((B,), jnp.float32),
        scratch_types=[
            pltpu.VMEM((bw,), jnp.int32),
            pltpu.VMEM((bw,), jnp.int32),
            pltpu.VMEM((bw, K), jnp.float32),
            pltpu.VMEM((bw, K), jnp.float32),
            pltpu.VMEM((bw + L,), jnp.float32),
            pltpu.SemaphoreType.DMA,
            pltpu.SemaphoreType.DMA,
        ],
        compiler_params=pltpu.CompilerParams(
            needs_layout_passes=False, use_tc_tiling_on_sc=False),
    )
    def mf(du, di, ut, it, out, idx_u, idx_i, u_rows, i_rows, out_v,
           sem_u, sem_i):
        wid = lax.axis_index("s") * NC + lax.axis_index("c")
        base = wid * bw

        pltpu.sync_copy(du.at[pl.ds(base, bw)], idx_u)
        pltpu.sync_copy(di.at[pl.ds(base, bw)], idx_i)

        copies = []
        for c in range(bw // CH):
            copies.append(pltpu.async_copy(
                ut.at[idx_u.at[pl.ds(c * CH, CH)]],
                u_rows.at[pl.ds(c * CH, CH)], sem_u))
            copies.append(pltpu.async_copy(
                it.at[idx_i.at[pl.ds(c * CH, CH)]],
                i_rows.at[pl.ds(c * CH, CH)], sem_i))
        for cp in copies:
            cp.wait()

        last_lane = lax.iota(jnp.int32, L) == (L - 1)

        @plsc.parallel_loop(0, bw, 1, unroll=8)
        def _(r):
            p = (u_rows[r, pl.ds(0, L)] * i_rows[r, pl.ds(0, L)] +
                 u_rows[r, pl.ds(L, L)] * i_rows[r, pl.ds(L, L)])
            s = plsc.cumsum(p)
            plsc.store_compressed(out_v.at[pl.ds(r, L)], s, mask=last_lane)

        pltpu.sync_copy(out_v.at[pl.ds(0, bw)], out.at[pl.ds(base, bw)])

    u_rm = _to_row_major(u_table.T)
    i_rm = _to_row_major(i_table.T)
    return mf(data_u.astype(jnp.int32), data_i.astype(jnp.int32), u_rm, i_rm)
